# baseline (speedup 1.0000x reference)
import functools

import jax
import jax.numpy as jnp
from jax import lax
from jax.experimental import pallas as pl
from jax.experimental.pallas import tpu as pltpu
from jax.experimental.pallas import tpu_sc as plsc

N_NODES = 10000
N_PAD = 10112
D = 128
DP = 144
E = 320000
NC, NS = 2, 16
NW = NC * NS
CHUNK = 128
NCHUNK = 80
E_PAD = NW * NCHUNK * CHUNK
NBUF = 1
ROWS_PER_TILE = N_PAD // NS


def _sc_aggregate(x_aug, src_r, dst_r, zeros_blk):
    mesh = plsc.VectorSubcoreMesh(core_axis_name="c", subcore_axis_name="s")

    @functools.partial(
        pl.kernel,
        out_type=jax.ShapeDtypeStruct((NC, N_PAD, DP), jnp.float32),
        mesh=mesh,
        compiler_params=pltpu.CompilerParams(use_tc_tiling_on_sc=False),
        scratch_types=[
            pltpu.VMEM_SHARED((N_PAD, DP), jnp.float32),
            pltpu.VMEM((NCHUNK, CHUNK), jnp.int32),
            pltpu.VMEM((NCHUNK, CHUNK), jnp.int32),
        ]
        + [pltpu.VMEM((CHUNK, DP), jnp.float32) for _ in range(NBUF)]
        + [pltpu.SemaphoreType.DMA for _ in range(NBUF)],
    )
    def k(x_hbm, src_hbm, dst_hbm, zeros_hbm, out_hbm,
          acc, src_v, dst_v, *bufs_sems):
        bufs = bufs_sems[:NBUF]
        sems = bufs_sems[NBUF:]
        c = lax.axis_index("c")
        s = lax.axis_index("s")
        w = s * NC + c
        pltpu.sync_copy(src_hbm.at[w], src_v)
        pltpu.sync_copy(dst_hbm.at[w], dst_v)
        pltpu.sync_copy(zeros_hbm, acc.at[pl.ds(s * ROWS_PER_TILE, ROWS_PER_TILE)])
        plsc.subcore_barrier()

        def step(j, carry):
            pltpu.async_copy(x_hbm.at[src_v.at[j]], bufs[0], sems[0]).wait()
            pltpu.sync_copy(bufs[0], acc.at[dst_v.at[j]], add=True)
            return carry

        lax.fori_loop(0, NCHUNK, step, 0)

        plsc.subcore_barrier()
        pltpu.sync_copy(
            acc.at[pl.ds(s * ROWS_PER_TILE, ROWS_PER_TILE)],
            out_hbm.at[c, pl.ds(s * ROWS_PER_TILE, ROWS_PER_TILE)],
        )

    return k(x_aug, src_r, dst_r, zeros_blk)


BLK = 1000


def _tc_tail(parts, x, W_l, W_r, b_l2d):
    def body(p0_ref, p1_ref, x_ref, wl_ref, wr_ref, b_ref, o_ref):
        p = p0_ref[0] + p1_ref[0]
        agg = p[:, :D]
        deg = p[:, D:D + 1]
        mean = agg / jnp.maximum(deg, 1.0)
        h = (jnp.dot(mean, wl_ref[...], preferred_element_type=jnp.float32)
             + b_ref[...]
             + jnp.dot(x_ref[...], wr_ref[...], preferred_element_type=jnp.float32))
        h = jnp.maximum(h, 0.0)
        n = jnp.sqrt(jnp.sum(h * h, axis=1, keepdims=True))
        o_ref[...] = h / (n + 1e-9)

    return pl.pallas_call(
        body,
        grid=(N_NODES // BLK,),
        in_specs=[
            pl.BlockSpec((1, BLK, DP), lambda i: (0, i, 0)),
            pl.BlockSpec((1, BLK, DP), lambda i: (1, i, 0)),
            pl.BlockSpec((BLK, D), lambda i: (i, 0)),
            pl.BlockSpec((D, D), lambda i: (0, 0)),
            pl.BlockSpec((D, D), lambda i: (0, 0)),
            pl.BlockSpec((1, D), lambda i: (0, 0)),
        ],
        out_specs=pl.BlockSpec((BLK, D), lambda i: (i, 0)),
        out_shape=jax.ShapeDtypeStruct((N_NODES, D), jnp.float32),
    )(parts, parts, x, W_l, W_r, b_l2d)


def kernel(x, edge_index, W_l, W_r, b_l):
    ei = edge_index.astype(jnp.int32)
    pad_idx = jnp.full((2, E_PAD - E), N_NODES, jnp.int32)
    ei = jnp.concatenate([ei, pad_idx], axis=1)
    src_r = ei[0].reshape(NW, NCHUNK, CHUNK)
    dst_r = ei[1].reshape(NW, NCHUNK, CHUNK)
    x_aug = jnp.concatenate(
        [x,
         jnp.ones((N_NODES, 1), jnp.float32),
         jnp.zeros((N_NODES, DP - D - 1), jnp.float32)],
        axis=1,
    )
    x_aug = jnp.concatenate(
        [x_aug, jnp.zeros((N_PAD - N_NODES, DP), jnp.float32)], axis=0)
    zeros_blk = jnp.zeros((ROWS_PER_TILE, DP), jnp.float32)
    parts = _sc_aggregate(x_aug, src_r, dst_r, zeros_blk)
    return _tc_tail(parts, x, W_l, W_r, b_l.reshape(1, D))


# speedup vs baseline: 2.6842x; 2.6842x over previous
import functools

import jax
import jax.numpy as jnp
from jax import lax
from jax.experimental import pallas as pl
from jax.experimental.pallas import tpu as pltpu
from jax.experimental.pallas import tpu_sc as plsc

N_NODES = 10000
N_PAD = 10112
D = 128
DP = 144
E = 320000
NC, NS = 2, 16
NW = NC * NS
CHUNK = 80
NCHUNK = 125
NBUF = 2
NIB = 4
ROWS_PER_TILE = N_PAD // NS


def _sc_aggregate(x_aug, src_r, dst_r, zeros_blk):
    mesh = plsc.VectorSubcoreMesh(core_axis_name="c", subcore_axis_name="s")

    @functools.partial(
        pl.kernel,
        out_type=jax.ShapeDtypeStruct((NC, N_PAD, DP), jnp.float32),
        mesh=mesh,
        compiler_params=pltpu.CompilerParams(use_tc_tiling_on_sc=False),
        scratch_types=[
            pltpu.VMEM_SHARED((N_PAD, DP), jnp.float32),
            pltpu.VMEM((NCHUNK, CHUNK), jnp.int32),
        ]
        + [pltpu.VMEM((CHUNK,), jnp.int32) for _ in range(NIB)]
        + [pltpu.VMEM((CHUNK, DP), jnp.float32) for _ in range(NBUF)]
        + [pltpu.SemaphoreType.DMA for _ in range(NIB + NBUF)],
    )
    def k(x_hbm, src_hbm, dst_hbm, zeros_hbm, out_hbm,
          acc, dst_v, *bufs_sems):
        sbuf = bufs_sems[:NIB]
        rows = bufs_sems[NIB:NIB + NBUF]
        isem = bufs_sems[NIB + NBUF:2 * NIB + NBUF]
        gsem = bufs_sems[2 * NIB + NBUF:]
        c = lax.axis_index("c")
        s = lax.axis_index("s")
        w = s * NC + c
        pltpu.sync_copy(dst_hbm.at[w], dst_v)
        pltpu.sync_copy(zeros_hbm, acc.at[pl.ds(s * ROWS_PER_TILE, ROWS_PER_TILE)])
        for t in range(NIB):
            pltpu.async_copy(src_hbm.at[w, t], sbuf[t], isem[t])
        for t in range(NBUF):
            pltpu.make_async_copy(src_hbm.at[w, t], sbuf[t], isem[t]).wait()
            pltpu.async_copy(x_hbm.at[sbuf[t]], rows[t], gsem[t])
        plsc.subcore_barrier()

        def chunk_step(j, b, fire_idx, fire_gather):
            rb = b % NBUF
            pltpu.make_async_copy(x_hbm.at[sbuf[b]], rows[rb], gsem[rb]).wait()
            pltpu.sync_copy(rows[rb], acc.at[dst_v.at[j]], add=True)
            if fire_idx:
                pltpu.async_copy(src_hbm.at[w, j + NIB], sbuf[b], isem[b])
            if fire_gather:
                b2 = (b + NBUF) % NIB
                pltpu.make_async_copy(src_hbm.at[w, j + NBUF], sbuf[b2], isem[b2]).wait()
                pltpu.async_copy(x_hbm.at[sbuf[b2]], rows[rb], gsem[rb])

        def ring(g, carry):
            for b in range(NIB):
                chunk_step(g * NIB + b, b, True, True)
            return carry

        lax.fori_loop(0, (NCHUNK - NIB - 1) // NIB, ring, 0)
        chunk_step(120, 0, True, True)
        chunk_step(121, 1, False, True)
        chunk_step(122, 2, False, True)
        chunk_step(123, 3, False, False)
        chunk_step(124, 0, False, False)

        plsc.subcore_barrier()
        pltpu.sync_copy(
            acc.at[pl.ds(s * ROWS_PER_TILE, ROWS_PER_TILE)],
            out_hbm.at[c, pl.ds(s * ROWS_PER_TILE, ROWS_PER_TILE)],
        )

    return k(x_aug, src_r, dst_r, zeros_blk)


BLK = 1000


def _tc_tail(parts, x, W_l, W_r, b_l2d):
    def body(p0_ref, p1_ref, x_ref, wl_ref, wr_ref, b_ref, o_ref):
        p = p0_ref[0] + p1_ref[0]
        agg = p[:, :D]
        deg = p[:, D:D + 1]
        mean = agg / jnp.maximum(deg, 1.0)
        h = (jnp.dot(mean, wl_ref[...], preferred_element_type=jnp.float32)
             + b_ref[...]
             + jnp.dot(x_ref[...], wr_ref[...], preferred_element_type=jnp.float32))
        h = jnp.maximum(h, 0.0)
        n = jnp.sqrt(jnp.sum(h * h, axis=1, keepdims=True))
        o_ref[...] = h / (n + 1e-9)

    return pl.pallas_call(
        body,
        grid=(N_NODES // BLK,),
        in_specs=[
            pl.BlockSpec((1, BLK, DP), lambda i: (0, i, 0)),
            pl.BlockSpec((1, BLK, DP), lambda i: (1, i, 0)),
            pl.BlockSpec((BLK, D), lambda i: (i, 0)),
            pl.BlockSpec((D, D), lambda i: (0, 0)),
            pl.BlockSpec((D, D), lambda i: (0, 0)),
            pl.BlockSpec((1, D), lambda i: (0, 0)),
        ],
        out_specs=pl.BlockSpec((BLK, D), lambda i: (i, 0)),
        out_shape=jax.ShapeDtypeStruct((N_NODES, D), jnp.float32),
    )(parts, parts, x, W_l, W_r, b_l2d)


def kernel(x, edge_index, W_l, W_r, b_l):
    ei = edge_index.astype(jnp.int32)
    src_r = ei[0].reshape(NW, NCHUNK, CHUNK)
    dst_r = ei[1].reshape(NW, NCHUNK, CHUNK)
    x_aug = jnp.concatenate(
        [x,
         jnp.ones((N_NODES, 1), jnp.float32),
         jnp.zeros((N_NODES, DP - D - 1), jnp.float32)],
        axis=1,
    )
    zeros_blk = jnp.zeros((ROWS_PER_TILE, DP), jnp.float32)
    parts = _sc_aggregate(x_aug, src_r, dst_r, zeros_blk)
    return _tc_tail(parts, x, W_l, W_r, b_l.reshape(1, D))
